# baseline (speedup 1.0000x reference)
import functools

import jax
import jax.numpy as jnp
from jax import lax
from jax.experimental import pallas as pl
from jax.experimental.pallas import tpu as pltpu
from jax.experimental.pallas import tpu_sc as plsc

O = 4096
I = 4096
TOTAL_WORDS = O * I
HALF = TOTAL_WORDS // 2

NC = 2
NS = 16
LANES = 16

CHUNK_WORDS = 1769472
NUM_CHUNKS = -(-HALF // CHUNK_WORDS)
LAST_WORDS = HALF - (NUM_CHUNKS - 1) * CHUNK_WORDS

ROUND = 2048
UNROLL = 16
BLK = 64
BLK_SHIFT = 6
RB = 64
RB_SHIFT = 6
STAGE = ROUND + RB + LANES
SENTINEL = 1 << 28


def _sc_body(nnz_pad, runcap, f_hbm, v_hbm, wt_hbm, ri_hbm, rf_hbm,
             fbuf0, fbuf1, vbuf0, vbuf1, ibuf0, ibuf1, sbuf0, sbuf1, zbuf,
             shared, zsem, lsf0, lsf1, lsv0, lsv1, fsem0, fsem1):
    c = lax.axis_index("c")
    s = lax.axis_index("s")
    seg = nnz_pad // NS
    rounds0 = seg // ROUND
    seg_base = s * seg
    runbase = (c * NS + s) * runcap
    iota = lax.broadcasted_iota(jnp.int32, (LANES,), 0)
    zeros_i = jnp.zeros((LANES,), jnp.int32)
    zeros_f = jnp.zeros((LANES,), jnp.float32)
    sent_i = jnp.full((LANES,), SENTINEL, jnp.int32)

    def _z(i, _):
        zbuf[pl.ds(i * LANES, LANES)] = zeros_f
        return 0
    lax.fori_loop(0, ROUND // LANES, _z, 0)

    def _p0_load_start(r, fb, vb, sf, sv):
        off = seg_base + r * ROUND
        pltpu.async_copy(f_hbm.at[pl.ds(off, ROUND)], fb, sf)
        pltpu.async_copy(v_hbm.at[pl.ds(off, ROUND)], vb, sv)

    def _p0_load_wait(r, fb, vb, sf, sv):
        off = seg_base + r * ROUND
        pltpu.make_async_copy(f_hbm.at[pl.ds(off, ROUND)], fb, sf).wait()
        pltpu.make_async_copy(v_hbm.at[pl.ds(off, ROUND)], vb, sv).wait()

    def _run_drain(pend, ibuf, sbuf, fsem):
        def _w(j, _):
            sl = pl.ds(0, RB)
            dst = pl.ds(runbase, RB)
            pltpu.make_async_copy(ibuf.at[sl], ri_hbm.at[dst], fsem).wait()
            pltpu.make_async_copy(sbuf.at[sl], rf_hbm.at[dst], fsem).wait()
            return 0
        lax.fori_loop(0, pend, _w, 0)

    coff = c * HALF

    def _p0_half(r, fb, vb, ibuf, sbuf, sf, sv, fsem, pend, written):
        written = pl.multiple_of(written, RB)
        _run_drain(pend, ibuf, sbuf, fsem)
        _p0_load_wait(r, fb, vb, sf, sv)

        def _filt(i, cnt):
            parts = []
            for u in range(UNROLL):
                sl = pl.ds((i * UNROLL + u) * LANES, LANES)
                fv = fb[sl]
                m = ((fv >> 23) & 1) == c
                lh = fv - coff
                pos = plsc.cumsum(m.astype(jnp.int32))
                pc = plsc.all_reduce_population_count(m)
                parts.append((sl, lh, m, pos, pc))
            for sl, lh, m, pos, pc in parts:
                dst = cnt + pos - 1
                plsc.store_scatter(ibuf, [dst], lh, mask=m)
                plsc.store_scatter(sbuf, [dst], vb[sl], mask=m)
                cnt = cnt + pc
            return cnt
        cnt = lax.fori_loop(0, ROUND // (LANES * UNROLL), _filt,
                            jnp.zeros((LANES,), jnp.int32))
        scnt = jnp.max(cnt)

        for j in range(RB // LANES):
            pad = scnt + j * LANES + iota
            plsc.store_scatter(ibuf, [pad], sent_i)
            plsc.store_scatter(sbuf, [pad], zeros_f)

        nrb = (scnt + RB - 1) >> RB_SHIFT

        def _fl(j, _):
            sl = pl.ds(j * RB, RB)
            dst = pl.ds(runbase + written + j * RB, RB)
            pltpu.async_copy(ibuf.at[sl], ri_hbm.at[dst], fsem)
            pltpu.async_copy(sbuf.at[sl], rf_hbm.at[dst], fsem)
            return 0
        lax.fori_loop(0, nrb, _fl, 0)

        @pl.when(r + 2 < rounds0)
        def _prefetch():
            _p0_load_start(r + 2, fb, vb, sf, sv)
        return nrb, written + (nrb << RB_SHIFT)

    _p0_load_start(0, fbuf0, vbuf0, lsf0, lsv0)
    _p0_load_start(1, fbuf1, vbuf1, lsf1, lsv1)

    def _p0_pair(t, carry):
        pend0, pend1, written = carry
        pend0, written = _p0_half(2 * t, fbuf0, vbuf0, ibuf0, sbuf0,
                                  lsf0, lsv0, fsem0, pend0, written)
        pend1, written = _p0_half(2 * t + 1, fbuf1, vbuf1, ibuf1, sbuf1,
                                  lsf1, lsv1, fsem1, pend1, written)
        return pend0, pend1, written
    pend0, pend1, written = lax.fori_loop(
        0, rounds0 // 2, _p0_pair,
        (jnp.int32(0), jnp.int32(0), jnp.int32(0)))
    _run_drain(pend0, ibuf0, sbuf0, fsem0)
    _run_drain(pend1, ibuf1, sbuf1, fsem1)

    def _sent(i, _):
        ibuf0[pl.ds(i * LANES, LANES)] = sent_i
        sbuf0[pl.ds(i * LANES, LANES)] = zeros_f
        return 0
    lax.fori_loop(0, RB // LANES, _sent, 0)
    target = ((written + 2 * ROUND - 1) // (2 * ROUND)) * (2 * ROUND)

    written8 = pl.multiple_of(written, RB)

    def _pb(j, _):
        dst = pl.ds(runbase + written8 + j * RB, RB)
        pltpu.sync_copy(ibuf0.at[pl.ds(0, RB)], ri_hbm.at[dst])
        pltpu.sync_copy(sbuf0.at[pl.ds(0, RB)], rf_hbm.at[dst])
        return 0
    lax.fori_loop(0, (target - written) >> RB_SHIFT, _pb, 0)
    rounds_t = target // ROUND
    rounds_t = pl.multiple_of(rounds_t, 2)

    def _load_start(r, fb, vb, sf, sv):
        off = runbase + r * ROUND
        pltpu.async_copy(ri_hbm.at[pl.ds(off, ROUND)], fb, sf)
        pltpu.async_copy(rf_hbm.at[pl.ds(off, ROUND)], vb, sv)

    def _load_wait(r, fb, vb, sf, sv):
        off = runbase + r * ROUND
        pltpu.make_async_copy(ri_hbm.at[pl.ds(off, ROUND)], fb, sf).wait()
        pltpu.make_async_copy(rf_hbm.at[pl.ds(off, ROUND)], vb, sv).wait()

    def _drain(pend, sbuf, ibuf, fsem):
        def _w(j, _):
            sl = pl.ds(j * BLK, BLK)
            pltpu.make_async_copy(sbuf.at[sl], shared.at[ibuf.at[sl]],
                                  fsem).wait()
            return 0
        lax.fori_loop(0, pend, _w, 0)

    bound = plsc.bitcast(jnp.full((LANES,), CHUNK_WORDS, jnp.int32),
                         jnp.uint32)

    def _half_round(r, base, fb, vb, ibuf, sbuf, sf, sv, fsem, pend):
        _drain(pend, sbuf, ibuf, fsem)
        _load_wait(r, fb, vb, sf, sv)

        def _filt(i, cnt):
            parts = []
            for u in range(UNROLL):
                sl = pl.ds((i * UNROLL + u) * LANES, LANES)
                lf = fb[sl] - base
                m = plsc.bitcast(lf, jnp.uint32) < bound
                pos = plsc.cumsum(m.astype(jnp.int32))
                pc = plsc.all_reduce_population_count(m)
                parts.append((sl, lf, m, pos, pc))
            for sl, lf, m, pos, pc in parts:
                dst = cnt + pos - 1
                plsc.store_scatter(ibuf, [dst], lf, mask=m)
                plsc.store_scatter(sbuf, [dst], vb[sl], mask=m)
                cnt = cnt + pc
            return cnt
        cnt = lax.fori_loop(0, ROUND // (LANES * UNROLL), _filt,
                            jnp.zeros((LANES,), jnp.int32))
        scnt = jnp.max(cnt)

        for j in range(BLK // LANES):
            pad = scnt + j * LANES + iota
            plsc.store_scatter(ibuf, [pad], zeros_i)
            plsc.store_scatter(sbuf, [pad], zeros_f)

        nblk = (scnt + BLK - 1) >> BLK_SHIFT

        def _flush(j, _):
            sl = pl.ds(j * BLK, BLK)
            pltpu.async_copy(sbuf.at[sl], shared.at[ibuf.at[sl]], fsem,
                             add=True)
            return 0
        lax.fori_loop(0, nblk, _flush, 0)

        @pl.when(r + 2 < rounds_t)
        def _prefetch():
            _load_start(r + 2, fb, vb, sf, sv)
        return nblk

    def _chunk_body(base, cw, wb):
        nz = wb // ROUND
        for j in range(nz):
            pltpu.async_copy(zbuf, shared.at[pl.ds(s * wb + j * ROUND,
                                                   ROUND)], zsem)
        for j in range(nz):
            pltpu.make_async_copy(zbuf, shared.at[pl.ds(s * wb + j * ROUND,
                                                        ROUND)], zsem).wait()
        plsc.subcore_barrier()

        @pl.when(rounds_t > 0)
        def _prime():
            _load_start(0, fbuf0, vbuf0, lsf0, lsv0)
            _load_start(1, fbuf1, vbuf1, lsf1, lsv1)

        def _pair(t, carry):
            pend0, pend1 = carry
            pend0 = _half_round(2 * t, base, fbuf0, vbuf0, ibuf0, sbuf0,
                                lsf0, lsv0, fsem0, pend0)
            pend1 = _half_round(2 * t + 1, base, fbuf1, vbuf1, ibuf1, sbuf1,
                                lsf1, lsv1, fsem1, pend1)
            return pend0, pend1
        pend0, pend1 = lax.fori_loop(0, rounds_t // 2, _pair,
                                     (jnp.int32(0), jnp.int32(0)))
        _drain(pend0, sbuf0, ibuf0, fsem0)
        _drain(pend1, sbuf1, ibuf1, fsem1)
        plsc.subcore_barrier()

        pltpu.sync_copy(shared.at[pl.ds(s * wb, wb)],
                        wt_hbm.at[pl.ds(coff + base + s * wb, wb)])
        plsc.subcore_barrier()

    def _full_chunk(i, _):
        _chunk_body(i * CHUNK_WORDS, CHUNK_WORDS, CHUNK_WORDS // NS)
        return 0
    lax.fori_loop(0, NUM_CHUNKS - 1, _full_chunk, 0)
    _chunk_body((NUM_CHUNKS - 1) * CHUNK_WORDS, LAST_WORDS, LAST_WORDS // NS)


def _sc_scatter(f_pad, v_pad):
    nnz_pad = f_pad.shape[0]
    seg = nnz_pad // NS
    runcap = seg + 2 * ROUND
    mesh = plsc.VectorSubcoreMesh(core_axis_name="c", subcore_axis_name="s")
    return pl.kernel(
        functools.partial(_sc_body, nnz_pad, runcap),
        out_type=(
            jax.ShapeDtypeStruct((TOTAL_WORDS,), jnp.float32),
            jax.ShapeDtypeStruct((NC * NS * runcap,), jnp.int32),
            jax.ShapeDtypeStruct((NC * NS * runcap,), jnp.float32),
        ),
        mesh=mesh,
        compiler_params=pltpu.CompilerParams(needs_layout_passes=False),
        scratch_types=[
            pltpu.VMEM((ROUND,), jnp.int32),
            pltpu.VMEM((ROUND,), jnp.int32),
            pltpu.VMEM((ROUND,), jnp.float32),
            pltpu.VMEM((ROUND,), jnp.float32),
            pltpu.VMEM((STAGE,), jnp.int32),
            pltpu.VMEM((STAGE,), jnp.int32),
            pltpu.VMEM((STAGE,), jnp.float32),
            pltpu.VMEM((STAGE,), jnp.float32),
            pltpu.VMEM((ROUND,), jnp.float32),
            pltpu.VMEM_SHARED((CHUNK_WORDS,), jnp.float32),
            pltpu.SemaphoreType.DMA,
            pltpu.SemaphoreType.DMA,
            pltpu.SemaphoreType.DMA,
            pltpu.SemaphoreType.DMA,
            pltpu.SemaphoreType.DMA,
            pltpu.SemaphoreType.DMA,
            pltpu.SemaphoreType.DMA,
        ],
    )(f_pad, v_pad)


BN = 512


def _mm_body(x_ref, wt_ref, o_ref):
    o_ref[...] = jnp.dot(x_ref[...], wt_ref[...],
                         preferred_element_type=jnp.float32)


def _matmul(x, wt):
    b = x.shape[0]
    return pl.pallas_call(
        _mm_body,
        grid=(O // BN,),
        in_specs=[
            pl.BlockSpec((b, I), lambda j: (0, 0)),
            pl.BlockSpec((I, BN), lambda j: (0, j)),
        ],
        out_specs=pl.BlockSpec((b, BN), lambda j: (0, j)),
        out_shape=jax.ShapeDtypeStruct((b, O), jnp.float32),
    )(x, wt)


def kernel(x, rows, cols, values):
    nnz = rows.shape[0]
    f = cols * O + rows
    per_tile = NS * ROUND * 2
    nnz_pad = ((nnz + per_tile - 1) // per_tile) * per_tile
    pad = nnz_pad - nnz
    f_pad = jnp.concatenate([f, jnp.full((pad,), SENTINEL, jnp.int32)])
    v_pad = jnp.concatenate([values, jnp.zeros((pad,), jnp.float32)])
    wt_flat, _, _ = _sc_scatter(f_pad, v_pad)
    wt = wt_flat.reshape(I, O)
    return _matmul(x, wt)


# speedup vs baseline: 1.0144x; 1.0144x over previous
import functools

import jax
import jax.numpy as jnp
from jax import lax
from jax.experimental import pallas as pl
from jax.experimental.pallas import tpu as pltpu
from jax.experimental.pallas import tpu_sc as plsc

O = 4096
I = 4096
TOTAL_WORDS = O * I
HALF = TOTAL_WORDS // 2

NC = 2
NS = 16
LANES = 16

CHUNK_WORDS = 1769472
NUM_CHUNKS = -(-HALF // CHUNK_WORDS)
LAST_WORDS = HALF - (NUM_CHUNKS - 1) * CHUNK_WORDS

ROUND = 2048
UNROLL = 16
BLK = 64
BLK_SHIFT = 6
RB = 128
RB_SHIFT = 7
STAGE = ROUND + RB + LANES
SENTINEL = 1 << 28


def _sc_body(nnz_pad, runcap, f_hbm, v_hbm, wt_hbm, ri_hbm, rf_hbm,
             fbuf0, fbuf1, vbuf0, vbuf1, ibuf0, ibuf1, sbuf0, sbuf1, zbuf,
             shared, zsem, lsf0, lsf1, lsv0, lsv1, fsem0, fsem1):
    c = lax.axis_index("c")
    s = lax.axis_index("s")
    seg = nnz_pad // NS
    rounds0 = seg // ROUND
    seg_base = s * seg
    runbase = (c * NS + s) * runcap
    iota = lax.broadcasted_iota(jnp.int32, (LANES,), 0)
    zeros_i = jnp.zeros((LANES,), jnp.int32)
    zeros_f = jnp.zeros((LANES,), jnp.float32)
    sent_i = jnp.full((LANES,), SENTINEL, jnp.int32)

    def _z(i, _):
        zbuf[pl.ds(i * LANES, LANES)] = zeros_f
        return 0
    lax.fori_loop(0, ROUND // LANES, _z, 0)

    def _p0_load_start(r, fb, vb, sf, sv):
        off = seg_base + r * ROUND
        pltpu.async_copy(f_hbm.at[pl.ds(off, ROUND)], fb, sf)
        pltpu.async_copy(v_hbm.at[pl.ds(off, ROUND)], vb, sv)

    def _p0_load_wait(r, fb, vb, sf, sv):
        off = seg_base + r * ROUND
        pltpu.make_async_copy(f_hbm.at[pl.ds(off, ROUND)], fb, sf).wait()
        pltpu.make_async_copy(v_hbm.at[pl.ds(off, ROUND)], vb, sv).wait()

    def _run_drain(pend, ibuf, sbuf, fsem):
        def _w(j, _):
            sl = pl.ds(0, RB)
            dst = pl.ds(runbase, RB)
            pltpu.make_async_copy(ibuf.at[sl], ri_hbm.at[dst], fsem).wait()
            pltpu.make_async_copy(sbuf.at[sl], rf_hbm.at[dst], fsem).wait()
            return 0
        lax.fori_loop(0, pend, _w, 0)

    coff = c * HALF

    def _p0_half(r, fb, vb, ibuf, sbuf, sf, sv, fsem, pend, written):
        written = pl.multiple_of(written, RB)
        _run_drain(pend, ibuf, sbuf, fsem)
        _p0_load_wait(r, fb, vb, sf, sv)

        def _filt(i, cnt):
            parts = []
            for u in range(UNROLL):
                sl = pl.ds((i * UNROLL + u) * LANES, LANES)
                fv = fb[sl]
                m = ((fv >> 23) & 1) == c
                lh = fv - coff
                pos = plsc.cumsum(m.astype(jnp.int32))
                pc = plsc.all_reduce_population_count(m)
                parts.append((sl, lh, m, pos, pc))
            for sl, lh, m, pos, pc in parts:
                dst = cnt + pos - 1
                plsc.store_scatter(ibuf, [dst], lh, mask=m)
                plsc.store_scatter(sbuf, [dst], vb[sl], mask=m)
                cnt = cnt + pc
            return cnt
        cnt = lax.fori_loop(0, ROUND // (LANES * UNROLL), _filt,
                            jnp.zeros((LANES,), jnp.int32))
        scnt = jnp.max(cnt)

        for j in range(RB // LANES):
            pad = scnt + j * LANES + iota
            plsc.store_scatter(ibuf, [pad], sent_i)
            plsc.store_scatter(sbuf, [pad], zeros_f)

        nrb = (scnt + RB - 1) >> RB_SHIFT

        def _fl(j, _):
            sl = pl.ds(j * RB, RB)
            dst = pl.ds(runbase + written + j * RB, RB)
            pltpu.async_copy(ibuf.at[sl], ri_hbm.at[dst], fsem)
            pltpu.async_copy(sbuf.at[sl], rf_hbm.at[dst], fsem)
            return 0
        lax.fori_loop(0, nrb, _fl, 0)

        @pl.when(r + 2 < rounds0)
        def _prefetch():
            _p0_load_start(r + 2, fb, vb, sf, sv)
        return nrb, written + (nrb << RB_SHIFT)

    _p0_load_start(0, fbuf0, vbuf0, lsf0, lsv0)
    _p0_load_start(1, fbuf1, vbuf1, lsf1, lsv1)

    def _p0_pair(t, carry):
        pend0, pend1, written = carry
        pend0, written = _p0_half(2 * t, fbuf0, vbuf0, ibuf0, sbuf0,
                                  lsf0, lsv0, fsem0, pend0, written)
        pend1, written = _p0_half(2 * t + 1, fbuf1, vbuf1, ibuf1, sbuf1,
                                  lsf1, lsv1, fsem1, pend1, written)
        return pend0, pend1, written
    pend0, pend1, written = lax.fori_loop(
        0, rounds0 // 2, _p0_pair,
        (jnp.int32(0), jnp.int32(0), jnp.int32(0)))
    _run_drain(pend0, ibuf0, sbuf0, fsem0)
    _run_drain(pend1, ibuf1, sbuf1, fsem1)

    def _sent(i, _):
        ibuf0[pl.ds(i * LANES, LANES)] = sent_i
        sbuf0[pl.ds(i * LANES, LANES)] = zeros_f
        return 0
    lax.fori_loop(0, RB // LANES, _sent, 0)
    target = ((written + 2 * ROUND - 1) // (2 * ROUND)) * (2 * ROUND)

    written8 = pl.multiple_of(written, RB)

    def _pb(j, _):
        dst = pl.ds(runbase + written8 + j * RB, RB)
        pltpu.sync_copy(ibuf0.at[pl.ds(0, RB)], ri_hbm.at[dst])
        pltpu.sync_copy(sbuf0.at[pl.ds(0, RB)], rf_hbm.at[dst])
        return 0
    lax.fori_loop(0, (target - written) >> RB_SHIFT, _pb, 0)
    rounds_t = target // ROUND
    rounds_t = pl.multiple_of(rounds_t, 2)

    def _load_start(r, fb, vb, sf, sv):
        off = runbase + r * ROUND
        pltpu.async_copy(ri_hbm.at[pl.ds(off, ROUND)], fb, sf)
        pltpu.async_copy(rf_hbm.at[pl.ds(off, ROUND)], vb, sv)

    def _load_wait(r, fb, vb, sf, sv):
        off = runbase + r * ROUND
        pltpu.make_async_copy(ri_hbm.at[pl.ds(off, ROUND)], fb, sf).wait()
        pltpu.make_async_copy(rf_hbm.at[pl.ds(off, ROUND)], vb, sv).wait()

    def _drain(pend, sbuf, ibuf, fsem):
        def _w(j, _):
            sl = pl.ds(j * BLK, BLK)
            pltpu.make_async_copy(sbuf.at[sl], shared.at[ibuf.at[sl]],
                                  fsem).wait()
            return 0
        lax.fori_loop(0, pend, _w, 0)

    bound = plsc.bitcast(jnp.full((LANES,), CHUNK_WORDS, jnp.int32),
                         jnp.uint32)

    def _half_round(r, base, fb, vb, ibuf, sbuf, sf, sv, fsem, pend):
        _drain(pend, sbuf, ibuf, fsem)
        _load_wait(r, fb, vb, sf, sv)

        def _filt(i, cnt):
            parts = []
            for u in range(UNROLL):
                sl = pl.ds((i * UNROLL + u) * LANES, LANES)
                lf = fb[sl] - base
                m = plsc.bitcast(lf, jnp.uint32) < bound
                pos = plsc.cumsum(m.astype(jnp.int32))
                pc = plsc.all_reduce_population_count(m)
                parts.append((sl, lf, m, pos, pc))
            for sl, lf, m, pos, pc in parts:
                dst = cnt + pos - 1
                plsc.store_scatter(ibuf, [dst], lf, mask=m)
                plsc.store_scatter(sbuf, [dst], vb[sl], mask=m)
                cnt = cnt + pc
            return cnt
        cnt = lax.fori_loop(0, ROUND // (LANES * UNROLL), _filt,
                            jnp.zeros((LANES,), jnp.int32))
        scnt = jnp.max(cnt)

        for j in range(BLK // LANES):
            pad = scnt + j * LANES + iota
            plsc.store_scatter(ibuf, [pad], zeros_i)
            plsc.store_scatter(sbuf, [pad], zeros_f)

        nblk = (scnt + BLK - 1) >> BLK_SHIFT

        def _flush(j, _):
            sl = pl.ds(j * BLK, BLK)
            pltpu.async_copy(sbuf.at[sl], shared.at[ibuf.at[sl]], fsem,
                             add=True)
            return 0
        lax.fori_loop(0, nblk, _flush, 0)

        @pl.when(r + 2 < rounds_t)
        def _prefetch():
            _load_start(r + 2, fb, vb, sf, sv)
        return nblk

    def _chunk_body(base, cw, wb):
        nz = wb // ROUND
        for j in range(nz):
            pltpu.async_copy(zbuf, shared.at[pl.ds(s * wb + j * ROUND,
                                                   ROUND)], zsem)
        for j in range(nz):
            pltpu.make_async_copy(zbuf, shared.at[pl.ds(s * wb + j * ROUND,
                                                        ROUND)], zsem).wait()
        plsc.subcore_barrier()

        @pl.when(rounds_t > 0)
        def _prime():
            _load_start(0, fbuf0, vbuf0, lsf0, lsv0)
            _load_start(1, fbuf1, vbuf1, lsf1, lsv1)

        def _pair(t, carry):
            pend0, pend1 = carry
            pend0 = _half_round(2 * t, base, fbuf0, vbuf0, ibuf0, sbuf0,
                                lsf0, lsv0, fsem0, pend0)
            pend1 = _half_round(2 * t + 1, base, fbuf1, vbuf1, ibuf1, sbuf1,
                                lsf1, lsv1, fsem1, pend1)
            return pend0, pend1
        pend0, pend1 = lax.fori_loop(0, rounds_t // 2, _pair,
                                     (jnp.int32(0), jnp.int32(0)))
        _drain(pend0, sbuf0, ibuf0, fsem0)
        _drain(pend1, sbuf1, ibuf1, fsem1)
        plsc.subcore_barrier()

        pltpu.sync_copy(shared.at[pl.ds(s * wb, wb)],
                        wt_hbm.at[pl.ds(coff + base + s * wb, wb)])
        plsc.subcore_barrier()

    def _full_chunk(i, _):
        _chunk_body(i * CHUNK_WORDS, CHUNK_WORDS, CHUNK_WORDS // NS)
        return 0
    lax.fori_loop(0, NUM_CHUNKS - 1, _full_chunk, 0)
    _chunk_body((NUM_CHUNKS - 1) * CHUNK_WORDS, LAST_WORDS, LAST_WORDS // NS)


def _sc_scatter(f_pad, v_pad):
    nnz_pad = f_pad.shape[0]
    seg = nnz_pad // NS
    runcap = seg + 2 * ROUND
    mesh = plsc.VectorSubcoreMesh(core_axis_name="c", subcore_axis_name="s")
    return pl.kernel(
        functools.partial(_sc_body, nnz_pad, runcap),
        out_type=(
            jax.ShapeDtypeStruct((TOTAL_WORDS,), jnp.float32),
            jax.ShapeDtypeStruct((NC * NS * runcap,), jnp.int32),
            jax.ShapeDtypeStruct((NC * NS * runcap,), jnp.float32),
        ),
        mesh=mesh,
        compiler_params=pltpu.CompilerParams(needs_layout_passes=False),
        scratch_types=[
            pltpu.VMEM((ROUND,), jnp.int32),
            pltpu.VMEM((ROUND,), jnp.int32),
            pltpu.VMEM((ROUND,), jnp.float32),
            pltpu.VMEM((ROUND,), jnp.float32),
            pltpu.VMEM((STAGE,), jnp.int32),
            pltpu.VMEM((STAGE,), jnp.int32),
            pltpu.VMEM((STAGE,), jnp.float32),
            pltpu.VMEM((STAGE,), jnp.float32),
            pltpu.VMEM((ROUND,), jnp.float32),
            pltpu.VMEM_SHARED((CHUNK_WORDS,), jnp.float32),
            pltpu.SemaphoreType.DMA,
            pltpu.SemaphoreType.DMA,
            pltpu.SemaphoreType.DMA,
            pltpu.SemaphoreType.DMA,
            pltpu.SemaphoreType.DMA,
            pltpu.SemaphoreType.DMA,
            pltpu.SemaphoreType.DMA,
        ],
    )(f_pad, v_pad)


BN = 512


def _mm_body(x_ref, wt_ref, o_ref):
    o_ref[...] = jnp.dot(x_ref[...], wt_ref[...],
                         preferred_element_type=jnp.float32)


def _matmul(x, wt):
    b = x.shape[0]
    return pl.pallas_call(
        _mm_body,
        grid=(O // BN,),
        in_specs=[
            pl.BlockSpec((b, I), lambda j: (0, 0)),
            pl.BlockSpec((I, BN), lambda j: (0, j)),
        ],
        out_specs=pl.BlockSpec((b, BN), lambda j: (0, j)),
        out_shape=jax.ShapeDtypeStruct((b, O), jnp.float32),
    )(x, wt)


def kernel(x, rows, cols, values):
    nnz = rows.shape[0]
    f = cols * O + rows
    per_tile = NS * ROUND * 2
    nnz_pad = ((nnz + per_tile - 1) // per_tile) * per_tile
    pad = nnz_pad - nnz
    f_pad = jnp.concatenate([f, jnp.full((pad,), SENTINEL, jnp.int32)])
    v_pad = jnp.concatenate([values, jnp.zeros((pad,), jnp.float32)])
    wt_flat, _, _ = _sc_scatter(f_pad, v_pad)
    wt = wt_flat.reshape(I, O)
    return _matmul(x, wt)
